# baseline (speedup 1.0000x reference)
import jax
import jax.numpy as jnp
from jax import lax
from jax.experimental import pallas as pl
from jax.experimental.pallas import tpu as pltpu
from jax.experimental.pallas import tpu_sc as plsc

NC = 2
NS = 16
L = 16
D = 128
B = 32
G = NS * L
ZR = 40


def _segment_sum_sc(x, src, dst, C, nchunks):
    e_pad = src.shape[0] - B
    es = e_pad // NS
    nb = es // B
    rowsp = C // NS
    cpc = nchunks // NC
    n_pad = nchunks * C

    def body(src_hbm, dst_hbm, x_hbm, acc_hbm, deg_hbm,
             acc_s, deg_s, idx0, idx1, dv0, dv1, rb0, rb1, loc_v,
             ones_v, zrow_v, zdeg_v, deg_t, sem0, sem1):
        cid = lax.axis_index("c")
        sid = lax.axis_index("s")
        zero16 = jnp.zeros((L,), jnp.float32)
        one16 = jnp.ones((L,), jnp.float32)
        iota16 = lax.broadcasted_iota(jnp.int32, (L,), 0)
        garb = C + sid * L + iota16

        for k in range(B // L):
            ones_v[pl.ds(k * L, L)] = one16

        def zrow_body(r, _):
            for k in range(D // L):
                zrow_v[r, pl.ds(k * L, L)] = zero16
            return 0
        lax.fori_loop(0, ZR, zrow_body, 0)

        def zdeg_body(i, _):
            zdeg_v[pl.ds(i * L, L)] = zero16
            return 0
        lax.fori_loop(0, rowsp // L, zdeg_body, 0)

        eoff = sid * es

        def fire(b, idxv, dv, rb, sem):
            pltpu.sync_copy(src_hbm.at[pl.ds(eoff + b * B, B)], idxv)
            pltpu.sync_copy(dst_hbm.at[pl.ds(eoff + b * B, B)], dv)
            return pltpu.async_copy(x_hbm.at[idxv], rb, sem)

        def drain(rb, sem):
            pltpu.make_async_copy(x_hbm.at[pl.ds(0, B)], rb, sem).wait()

        def consume(base, dv, rb):
            for g in range(B // L):
                d = dv[pl.ds(g * L, L)]
                lo = d - base
                m = (lo >= 0) & (lo < C)
                loc_v[pl.ds(g * L, L)] = jnp.where(m, lo, garb)
            pltpu.sync_copy(rb, acc_s.at[loc_v], add=True)

        for ci in range(cpc):
            base = (ci * NC + cid) * C
            r0 = sid * rowsp
            for k in range(rowsp // ZR):
                pltpu.sync_copy(zrow_v, acc_s.at[pl.ds(r0 + k * ZR, ZR)])
            pltpu.sync_copy(zrow_v.at[pl.ds(0, L)],
                            acc_s.at[pl.ds(C + sid * L, L)])
            pltpu.sync_copy(zdeg_v, deg_s.at[pl.ds(r0, rowsp)])
            pltpu.sync_copy(zdeg_v.at[pl.ds(0, L)],
                            deg_s.at[pl.ds(C + sid * L, L)])
            plsc.subcore_barrier()

            fire(0, idx0, dv0, rb0, sem0)

            def pipe_body(kk, _):
                fire(2 * kk + 1, idx1, dv1, rb1, sem1)
                drain(rb0, sem0)
                consume(base, dv0, rb0)
                fire(2 * kk + 2, idx0, dv0, rb0, sem0)
                drain(rb1, sem1)
                consume(base, dv1, rb1)
                return 0
            lax.fori_loop(0, nb // 2, pipe_body, 0)
            drain(rb0, sem0)

            plsc.subcore_barrier()
            pltpu.sync_copy(acc_s.at[pl.ds(r0, rowsp)],
                            acc_hbm.at[pl.ds(base + r0, rowsp)])
            pltpu.sync_copy(deg_s.at[pl.ds(r0, rowsp)], deg_t)
            pltpu.sync_copy(deg_t, deg_hbm.at[pl.ds(base + r0, rowsp)])
            plsc.subcore_barrier()

    f = pl.kernel(
        body,
        out_type=(jax.ShapeDtypeStruct((n_pad, D), jnp.float32),
                  jax.ShapeDtypeStruct((n_pad,), jnp.float32)),
        mesh=plsc.VectorSubcoreMesh(core_axis_name="c", subcore_axis_name="s",
                                    num_cores=NC, num_subcores=NS),
        scratch_types=[
            pltpu.VMEM_SHARED((C + G, D), jnp.float32),
            pltpu.VMEM_SHARED((C + G,), jnp.float32),
            pltpu.VMEM((B,), jnp.int32),
            pltpu.VMEM((B,), jnp.int32),
            pltpu.VMEM((B,), jnp.int32),
            pltpu.VMEM((B,), jnp.int32),
            pltpu.VMEM((B, D), jnp.float32),
            pltpu.VMEM((B, D), jnp.float32),
            pltpu.VMEM((B,), jnp.int32),
            pltpu.VMEM((B,), jnp.float32),
            pltpu.VMEM((ZR, D), jnp.float32),
            pltpu.VMEM((rowsp,), jnp.float32),
            pltpu.VMEM((rowsp,), jnp.float32),
            pltpu.SemaphoreType.DMA,
            pltpu.SemaphoreType.DMA,
        ],
    )
    return f(src, dst, x)


def _pad_edges(src, dst, n_src, sentinel):
    e = src.shape[0]
    e_pad = -(-e // (NS * 2 * B)) * (NS * 2 * B)
    pad = e_pad + B - e
    psrc = (jnp.arange(pad, dtype=jnp.int32) * 61) % n_src
    src = jnp.concatenate([src, psrc])
    dst = jnp.concatenate([dst, jnp.full((pad,), sentinel, jnp.int32)])
    return src, dst


def _combine_tc(accs, degs, ws, bias, n_out):
    R = 1024
    nrel = len(accs)

    def body(*refs):
        acc_refs = refs[:nrel]
        deg_refs = refs[nrel:2 * nrel]
        w_refs = refs[2 * nrel:3 * nrel]
        b_ref = refs[3 * nrel]
        o_ref = refs[3 * nrel + 1]
        out = None
        for a, dg, w in zip(acc_refs, deg_refs, w_refs):
            inv = 1.0 / jnp.maximum(dg[...], 1.0)
            t = jnp.dot(a[...] * inv, w[...],
                        preferred_element_type=jnp.float32)
            out = t if out is None else out + t
        o_ref[...] = out + b_ref[...]

    in_specs = (
        [pl.BlockSpec((R, D), lambda i: (i, 0)) for _ in range(nrel)]
        + [pl.BlockSpec((R, 1), lambda i: (i, 0)) for _ in range(nrel)]
        + [pl.BlockSpec((D, D), lambda i: (0, 0)) for _ in range(nrel)]
        + [pl.BlockSpec((1, D), lambda i: (0, 0))]
    )
    return pl.pallas_call(
        body,
        grid=(pl.cdiv(n_out, R),),
        in_specs=in_specs,
        out_specs=pl.BlockSpec((R, D), lambda i: (i, 0)),
        out_shape=jax.ShapeDtypeStruct((n_out, D), jnp.float32),
    )(*accs, *[d.reshape(-1, 1) for d in degs], *ws, bias.reshape(1, D))


def kernel(x_author, x_institution, x_paper,
           writes_src, writes_dst, aff_src, aff_dst,
           cites_src, cites_dst, rev_writes_src, rev_writes_dst,
           rev_aff_src, rev_aff_dst, weight, bias):
    n_author, n_inst, n_paper = (x_author.shape[0], x_institution.shape[0],
                                 x_paper.shape[0])

    def run(x, src, dst, C, nchunks):
        s, d = _pad_edges(src, dst, x.shape[0], nchunks * C)
        return _segment_sum_sc(x, s, d, C, nchunks)

    acc0, deg0 = run(x_author, writes_src, writes_dst, 12800, 8)
    acc2, deg2 = run(x_paper, cites_src, cites_dst, 12800, 8)
    acc1, deg1 = run(x_author, aff_src, aff_dst, 5120, 2)
    acc3, deg3 = run(x_paper, rev_writes_src, rev_writes_dst, 12800, 4)
    acc4, deg4 = run(x_institution, rev_aff_src, rev_aff_dst, 12800, 4)

    paper = _combine_tc([acc0, acc2], [deg0, deg2],
                        [weight[0], weight[2]], bias, n_paper)
    inst = _combine_tc([acc1], [deg1], [weight[1]], bias, n_inst)
    author = _combine_tc([acc3, acc4], [deg3, deg4],
                         [weight[3], weight[4]], bias, n_author)
    return author, inst, paper


# speedup vs baseline: 1.4951x; 1.4951x over previous
import jax
import jax.numpy as jnp
from jax import lax
from jax.experimental import pallas as pl
from jax.experimental.pallas import tpu as pltpu
from jax.experimental.pallas import tpu_sc as plsc

NC = 2
NS = 16
L = 16
D = 128
B = 64
G = NS * L
ZR = 40


def _segment_sum_sc(x, src, dst, C, nchunks):
    e_pad = src.shape[0] - B
    es = e_pad // NS
    nb = es // B
    rowsp = C // NS
    cpc = nchunks // NC
    n_pad = nchunks * C

    def body(src_hbm, dst_hbm, x_hbm, acc_hbm, deg_hbm,
             acc_s, deg_s, idx0, idx1, dv0, dv1, rb0, rb1, loc_v,
             ones_v, zrow_v, zdeg_v, deg_t, sem0, sem1):
        cid = lax.axis_index("c")
        sid = lax.axis_index("s")
        zero16 = jnp.zeros((L,), jnp.float32)
        one16 = jnp.ones((L,), jnp.float32)
        iota16 = lax.broadcasted_iota(jnp.int32, (L,), 0)
        garb = C + sid * L + iota16

        for k in range(B // L):
            ones_v[pl.ds(k * L, L)] = one16

        def zrow_body(r, _):
            for k in range(D // L):
                zrow_v[r, pl.ds(k * L, L)] = zero16
            return 0
        lax.fori_loop(0, ZR, zrow_body, 0)

        def zdeg_body(i, _):
            zdeg_v[pl.ds(i * L, L)] = zero16
            return 0
        lax.fori_loop(0, rowsp // L, zdeg_body, 0)

        eoff = sid * es

        def fire(b, idxv, dv, rb, sem):
            pltpu.sync_copy(src_hbm.at[pl.ds(eoff + b * B, B)], idxv)
            pltpu.sync_copy(dst_hbm.at[pl.ds(eoff + b * B, B)], dv)
            return pltpu.async_copy(x_hbm.at[idxv], rb, sem)

        def drain(rb, sem):
            pltpu.make_async_copy(x_hbm.at[pl.ds(0, B)], rb, sem).wait()

        def consume(base, dv, rb):
            for g in range(B // L):
                d = dv[pl.ds(g * L, L)]
                lo = d - base
                m = (lo >= 0) & (lo < C)
                loc_v[pl.ds(g * L, L)] = jnp.where(m, lo, garb)
            pltpu.sync_copy(rb, acc_s.at[loc_v], add=True)
            pltpu.sync_copy(ones_v, deg_s.at[loc_v], add=True)

        for ci in range(cpc):
            base = (ci * NC + cid) * C
            r0 = sid * rowsp
            for k in range(rowsp // ZR):
                pltpu.sync_copy(zrow_v, acc_s.at[pl.ds(r0 + k * ZR, ZR)])
            pltpu.sync_copy(zrow_v.at[pl.ds(0, L)],
                            acc_s.at[pl.ds(C + sid * L, L)])
            pltpu.sync_copy(zdeg_v, deg_s.at[pl.ds(r0, rowsp)])
            pltpu.sync_copy(zdeg_v.at[pl.ds(0, L)],
                            deg_s.at[pl.ds(C + sid * L, L)])
            plsc.subcore_barrier()

            fire(0, idx0, dv0, rb0, sem0)

            def pipe_body(kk, _):
                fire(2 * kk + 1, idx1, dv1, rb1, sem1)
                drain(rb0, sem0)
                consume(base, dv0, rb0)
                fire(2 * kk + 2, idx0, dv0, rb0, sem0)
                drain(rb1, sem1)
                consume(base, dv1, rb1)
                return 0
            lax.fori_loop(0, nb // 2, pipe_body, 0)
            drain(rb0, sem0)

            plsc.subcore_barrier()
            pltpu.sync_copy(acc_s.at[pl.ds(r0, rowsp)],
                            acc_hbm.at[pl.ds(base + r0, rowsp)])
            pltpu.sync_copy(deg_s.at[pl.ds(r0, rowsp)], deg_t)
            pltpu.sync_copy(deg_t, deg_hbm.at[pl.ds(base + r0, rowsp)])
            plsc.subcore_barrier()

    f = pl.kernel(
        body,
        out_type=(jax.ShapeDtypeStruct((n_pad, D), jnp.float32),
                  jax.ShapeDtypeStruct((n_pad,), jnp.float32)),
        mesh=plsc.VectorSubcoreMesh(core_axis_name="c", subcore_axis_name="s",
                                    num_cores=NC, num_subcores=NS),
        scratch_types=[
            pltpu.VMEM_SHARED((C + G, D), jnp.float32),
            pltpu.VMEM_SHARED((C + G,), jnp.float32),
            pltpu.VMEM((B,), jnp.int32),
            pltpu.VMEM((B,), jnp.int32),
            pltpu.VMEM((B,), jnp.int32),
            pltpu.VMEM((B,), jnp.int32),
            pltpu.VMEM((B, D), jnp.float32),
            pltpu.VMEM((B, D), jnp.float32),
            pltpu.VMEM((B,), jnp.int32),
            pltpu.VMEM((B,), jnp.float32),
            pltpu.VMEM((ZR, D), jnp.float32),
            pltpu.VMEM((rowsp,), jnp.float32),
            pltpu.VMEM((rowsp,), jnp.float32),
            pltpu.SemaphoreType.DMA,
            pltpu.SemaphoreType.DMA,
        ],
    )
    return f(src, dst, x)


def _pad_edges(src, dst, n_src, sentinel):
    e = src.shape[0]
    e_pad = -(-e // (NS * 2 * B)) * (NS * 2 * B)
    pad = e_pad + B - e
    psrc = (jnp.arange(pad, dtype=jnp.int32) * 61) % n_src
    src = jnp.concatenate([src, psrc])
    dst = jnp.concatenate([dst, jnp.full((pad,), sentinel, jnp.int32)])
    return src, dst


def _combine_tc(accs, degs, ws, bias, n_out):
    R = 1024
    nrel = len(accs)

    def body(*refs):
        acc_refs = refs[:nrel]
        deg_refs = refs[nrel:2 * nrel]
        w_refs = refs[2 * nrel:3 * nrel]
        b_ref = refs[3 * nrel]
        o_ref = refs[3 * nrel + 1]
        out = None
        for a, dg, w in zip(acc_refs, deg_refs, w_refs):
            inv = 1.0 / jnp.maximum(dg[...], 1.0)
            t = jnp.dot(a[...] * inv, w[...],
                        preferred_element_type=jnp.float32)
            out = t if out is None else out + t
        o_ref[...] = out + b_ref[...]

    in_specs = (
        [pl.BlockSpec((R, D), lambda i: (i, 0)) for _ in range(nrel)]
        + [pl.BlockSpec((R, 1), lambda i: (i, 0)) for _ in range(nrel)]
        + [pl.BlockSpec((D, D), lambda i: (0, 0)) for _ in range(nrel)]
        + [pl.BlockSpec((1, D), lambda i: (0, 0))]
    )
    return pl.pallas_call(
        body,
        grid=(pl.cdiv(n_out, R),),
        in_specs=in_specs,
        out_specs=pl.BlockSpec((R, D), lambda i: (i, 0)),
        out_shape=jax.ShapeDtypeStruct((n_out, D), jnp.float32),
    )(*accs, *[d.reshape(-1, 1) for d in degs], *ws, bias.reshape(1, D))


def kernel(x_author, x_institution, x_paper,
           writes_src, writes_dst, aff_src, aff_dst,
           cites_src, cites_dst, rev_writes_src, rev_writes_dst,
           rev_aff_src, rev_aff_dst, weight, bias):
    n_author, n_inst, n_paper = (x_author.shape[0], x_institution.shape[0],
                                 x_paper.shape[0])

    def run(x, src, dst, C, nchunks):
        s, d = _pad_edges(src, dst, x.shape[0], nchunks * C)
        return _segment_sum_sc(x, s, d, C, nchunks)

    acc0, deg0 = run(x_author, writes_src, writes_dst, 12800, 8)
    acc2, deg2 = run(x_paper, cites_src, cites_dst, 12800, 8)
    acc1, deg1 = run(x_author, aff_src, aff_dst, 5120, 2)
    acc3, deg3 = run(x_paper, rev_writes_src, rev_writes_dst, 12800, 4)
    acc4, deg4 = run(x_institution, rev_aff_src, rev_aff_dst, 12800, 4)

    paper = _combine_tc([acc0, acc2], [deg0, deg2],
                        [weight[0], weight[2]], bias, n_paper)
    inst = _combine_tc([acc1], [deg1], [weight[1]], bias, n_inst)
    author = _combine_tc([acc3, acc4], [deg3, deg4],
                         [weight[3], weight[4]], bias, n_author)
    return author, inst, paper
